# baseline (speedup 1.0000x reference)
import functools

import jax
import jax.numpy as jnp
from jax import lax
from jax.experimental import pallas as pl
from jax.experimental.pallas import tpu as pltpu
from jax.experimental.pallas import tpu_sc as plsc

_NC = 2
_NS = 16
_NW = _NC * _NS
_L = 16


def _sc_scores(cw, xw, nw_flat, Wc2, Wx2, B, N, D):
    bpw = B // _NW
    CH = _L
    n_chunks = bpw // CH
    NIDX = CH * N
    DV = D // _L
    PR = 2 * D

    mesh = plsc.VectorSubcoreMesh(core_axis_name="c", subcore_axis_name="s")

    @functools.partial(
        pl.kernel, mesh=mesh,
        compiler_params=pltpu.CompilerParams(
            needs_layout_passes=False, use_tc_tiling_on_sc=False),
        out_type=(jax.ShapeDtypeStruct((B,), jnp.float32),
                  jax.ShapeDtypeStruct((B * N,), jnp.float32)),
        scratch_types=[
            pltpu.VMEM((CH,), jnp.int32),
            pltpu.VMEM((CH,), jnp.int32),
            pltpu.VMEM((NIDX,), jnp.int32),
            pltpu.VMEM((CH,), jnp.int32),
            pltpu.VMEM((CH,), jnp.int32),
            pltpu.VMEM((NIDX,), jnp.int32),
            pltpu.VMEM((CH, PR), jnp.float32),
            pltpu.VMEM((CH, PR), jnp.float32),
            pltpu.VMEM((NIDX, PR), jnp.float32),
            pltpu.VMEM((CH,), jnp.float32),
            pltpu.VMEM((NIDX,), jnp.float32),
            pltpu.SemaphoreType.DMA,
        ],
    )
    def k(cw_hbm, xw_hbm, nw_hbm, Wc_hbm, Wx_hbm, pos_hbm, neg_hbm,
          cidx, xidx, nidx, cpk, xpk, npk, cbuf, xbuf, nbuf, posb, negb, sem):
        wid = lax.axis_index("s") * _NC + lax.axis_index("c")
        base = wid * bpw
        lanes = lax.iota(jnp.int32, _L)

        def chunk_body(g, carry):
            goff = base + g * CH
            pltpu.sync_copy(cw_hbm.at[pl.ds(goff, CH)], cidx)
            pltpu.sync_copy(xw_hbm.at[pl.ds(goff, CH)], xidx)
            pltpu.sync_copy(nw_hbm.at[pl.ds(goff * N, NIDX)], nidx)
            for i in range(CH // _L):
                s = pl.ds(i * _L, _L)
                cpk[s] = lax.shift_right_logical(cidx[s], 1)
                xpk[s] = lax.shift_right_logical(xidx[s], 1)
            for i in range(NIDX // _L):
                s = pl.ds(i * _L, _L)
                npk[s] = lax.shift_right_logical(nidx[s], 1)
            cps = [pltpu.async_copy(Wc_hbm.at[cpk], cbuf, sem),
                   pltpu.async_copy(Wx_hbm.at[xpk], xbuf, sem)]
            j = 0
            while j < NIDX:
                w = min(128, NIDX - j)
                cps.append(pltpu.async_copy(
                    Wx_hbm.at[npk.at[pl.ds(j, w)]],
                    nbuf.at[pl.ds(j, w)], sem))
                j += w
            for cp in cps:
                cp.wait()

            offc_v = lax.shift_left(cidx[...] & 1, 6)
            offx_v = lax.shift_left(xidx[...] & 1, 6)
            offn_v = [lax.shift_left(nidx[pl.ds(i * _L, _L)] & 1, 6)
                      for i in range(NIDX // _L)]
            pv = jnp.zeros((_L,), jnp.float32)
            nvecs = [jnp.zeros((_L,), jnp.float32) for _ in range(NIDX // _L)]
            for e in range(CH):
                offc = offc_v[e]
                offx = offx_v[e]
                c = [cbuf[e, pl.ds(offc + k2 * _L, _L)] for k2 in range(DV)]
                x = [xbuf[e, pl.ds(offx + k2 * _L, _L)] for k2 in range(DV)]
                acc = c[0] * x[0]
                for k2 in range(1, DV):
                    acc = acc + c[k2] * x[k2]
                pv = jnp.where(lanes == e, jnp.sum(acc), pv)
                for n in range(N):
                    r = e * N + n
                    offn = offn_v[r // _L][r % _L]
                    y = [nbuf[r, pl.ds(offn + k2 * _L, _L)] for k2 in range(DV)]
                    a = c[0] * y[0]
                    for k2 in range(1, DV):
                        a = a + c[k2] * y[k2]
                    nvecs[r // _L] = jnp.where(
                        lanes == (r % _L), jnp.sum(a), nvecs[r // _L])
            posb[...] = pv
            for v in range(NIDX // _L):
                negb[pl.ds(v * _L, _L)] = nvecs[v]
            pltpu.sync_copy(posb, pos_hbm.at[pl.ds(goff, CH)])
            pltpu.sync_copy(negb, neg_hbm.at[pl.ds(goff * N, NIDX)])
            return carry

        lax.fori_loop(0, n_chunks, chunk_body, 0)

    return k(cw, xw, nw_flat, Wc2, Wx2)


def _tc_loss(pos2d, neg2d, B):
    def body(pos_ref, neg_ref, out_ref):
        def ls(x):
            return jnp.minimum(x, 0.0) - jnp.log1p(jnp.exp(-jnp.abs(x)))
        tot = jnp.sum(ls(pos_ref[...])) + jnp.sum(ls(-neg_ref[...]))
        out_ref[0, 0] = -tot / B

    return pl.pallas_call(
        body,
        out_shape=jax.ShapeDtypeStruct((1, 1), jnp.float32),
        out_specs=pl.BlockSpec(memory_space=pltpu.SMEM),
    )(pos2d, neg2d)


def kernel(center_words, context_words, negative_words, W_center, W_context):
    B, N = negative_words.shape
    V, D = W_center.shape
    cw = center_words.astype(jnp.int32)
    xw = context_words.astype(jnp.int32)
    nw = negative_words.astype(jnp.int32).reshape(B * N)
    Wc2 = W_center.reshape(V // 2, 2 * D)
    Wx2 = W_context.reshape(V // 2, 2 * D)
    pos, neg = _sc_scores(cw, xw, nw, Wc2, Wx2, B, N, D)
    loss = _tc_loss(pos.reshape(B // 128, 128), neg.reshape(B * N // 128, 128), B)
    return loss[0, 0]
